# baseline (speedup 1.0000x reference)
import functools
import math

import jax
import jax.numpy as jnp
import numpy as np
from jax import lax
from jax.experimental import pallas as pl
from jax.experimental.pallas import tpu as pltpu
from jax.experimental.pallas import tpu_sc as plsc

MAX_LEN = 5000
B, SEG, SL = 32, 16, 256
V, D = 100000, 128
PAD = 0

N = B * SEG * SL
S = SEG * SL
R, CL = 32, 128
SCALE = math.sqrt(D)

NC, NS, L = 2, 16, 16
NW = NC * NS
TOK_PER_W = N // NW
CHUNK = 64
NBUF = 4
NCHUNK = TOK_PER_W // CHUNK

PE_ROWS = 4224
PE_ZERO = S


def _build_pe() -> np.ndarray:
    pe = np.zeros((PE_ROWS, D), dtype=np.float32)
    position = np.arange(0, S, dtype=np.float32)[:, None]
    div_term = np.exp(
        np.arange(0, D, 2, dtype=np.float32) * -(math.log(10000.0) / D))
    pe[:S, 0::2] = np.sin(position * div_term)
    pe[:S, 1::2] = np.cos(position * div_term)
    return pe


def _pack_pe_bf16(pe: np.ndarray) -> np.ndarray:
    bits = pe.view(np.uint32)
    bf = ((bits + 0x7FFF + ((bits >> 16) & 1)) >> 16).astype(np.uint16)
    bfr = bf.reshape(PE_ROWS, D // 32, 2, 16)
    inter = np.empty((PE_ROWS, D // 32, 16, 2), np.uint16)
    inter[..., 0] = bfr[:, :, 0, :]
    inter[..., 1] = bfr[:, :, 1, :]
    return inter.reshape(PE_ROWS, D)


_PE_PAD = _build_pe()
_PE_BF16_BITS = _pack_pe_bf16(_PE_PAD)


def _incl_cumsum(x, axis):
    n = x.shape[axis]
    k = 1
    while k < n:
        shape = list(x.shape)
        shape[axis] = k
        z = jnp.zeros(shape, x.dtype)
        shifted = jnp.concatenate(
            [z, lax.slice_in_dim(x, 0, n - k, axis=axis)], axis=axis)
        x = x + shifted
        k *= 2
    return x


def _pe_index_body(idx_ref, out_ref):
    idx = idx_ref[...]
    m = (idx != PAD).astype(jnp.int32)

    inc_b = _incl_cumsum(m, axis=0)
    p_excl = inc_b - m
    cc = lax.index_in_dim(inc_b, B - 1, axis=0, keepdims=False)

    inc_c = _incl_cumsum(cc, axis=1)
    row_tot = lax.slice_in_dim(inc_c, CL - 1, CL, axis=1)
    inc_r = _incl_cumsum(row_tot, axis=0)
    s_excl = (inc_r - row_tot) + (inc_c - cc)

    rank = p_excl + s_excl[None, :, :]

    seq = jnp.sum(jnp.sum(m, axis=2), axis=1, keepdims=True)
    c_incl = _incl_cumsum(seq, axis=0)

    c_at = jnp.zeros_like(rank)
    for j in range(B):
        cj = lax.slice(c_incl, (j, 0), (j + 1, 1)).reshape(1, 1, 1)
        dj = lax.slice(seq, (j, 0), (j + 1, 1)).reshape(1, 1, 1)
        c_at = c_at + jnp.where(rank >= cj, dj, 0)

    offset = rank - c_at
    out_ref[...] = jnp.where(idx != PAD, offset, PE_ZERO)


_pe_index_call = pl.pallas_call(
    _pe_index_body,
    out_shape=jax.ShapeDtypeStruct((B, R, CL), jnp.int32),
)


def _compute_chunk(wr, pr, ob):
    def row(r, c):
        for j in range(D // L):
            w = wr[r, pl.ds(j * L, L)]
            p = pr[r, pl.ds(j * L, L)]
            ob[r, pl.ds(j * L, L)] = w * SCALE + p
        return c

    lax.fori_loop(0, CHUNK, row, 0, unroll=4)


def _sc_body(word_hbm, pe_hbm, idx_hbm, pidx_hbm, out_hbm,
             idx_all, pidx_all, wrows, prows, obuf, *sems):
    sem_w = sems[0:NBUF]
    sem_p = sems[NBUF:2 * NBUF]
    sem_o = sems[2 * NBUF:3 * NBUF]

    wid = lax.axis_index("s") * NC + lax.axis_index("c")
    base = pl.multiple_of(wid * TOK_PER_W, TOK_PER_W)
    pltpu.sync_copy(idx_hbm.at[pl.ds(base, TOK_PER_W)], idx_all)
    pltpu.sync_copy(pidx_hbm.at[pl.ds(base, TOK_PER_W)], pidx_all)

    def start_gathers(g, b):
        off = pl.multiple_of(g * CHUNK, CHUNK)
        pltpu.async_copy(
            word_hbm.at[idx_all.at[pl.ds(off, CHUNK)]], wrows.at[b], sem_w[b])
        pltpu.async_copy(
            pe_hbm.at[pidx_all.at[pl.ds(off, CHUNK)]], prows.at[b], sem_w[b])

    def wait_gathers(b):
        pltpu.make_async_copy(
            word_hbm.at[idx_all.at[pl.ds(0, CHUNK)]], wrows.at[b],
            sem_w[b]).wait()
        pltpu.make_async_copy(
            pe_hbm.at[pidx_all.at[pl.ds(0, CHUNK)]], prows.at[b],
            sem_w[b]).wait()

    def start_out(g, b):
        off = pl.multiple_of(base + g * CHUNK, CHUNK)
        pltpu.async_copy(obuf.at[b], out_hbm.at[pl.ds(off, CHUNK)], sem_o[b])

    def wait_out(b):
        pltpu.make_async_copy(
            obuf.at[b], out_hbm.at[pl.ds(0, CHUNK)], sem_o[b]).wait()

    for b in range(NBUF):
        start_gathers(b, b)

    def group(k, c):
        g0 = k * NBUF
        for b in range(NBUF):
            g = g0 + b
            wait_gathers(b)

            @pl.when(g >= NBUF)
            def _():
                wait_out(b)

            _compute_chunk(wrows.at[b], prows.at[b], obuf.at[b])
            start_out(g, b)

            @pl.when(g + NBUF < NCHUNK)
            def _():
                start_gathers(g + NBUF, b)

        return c

    lax.fori_loop(0, NCHUNK // NBUF, group, 0)
    for b in range(NBUF):
        wait_out(b)


@functools.cache
def _sc_call():
    return functools.partial(
        pl.kernel,
        out_type=jax.ShapeDtypeStruct((N, D), jnp.float32),
        mesh=plsc.VectorSubcoreMesh(
            core_axis_name="c", subcore_axis_name="s",
            num_cores=NC, num_subcores=NS),
        scratch_types=[
            pltpu.VMEM((TOK_PER_W,), jnp.int32),
            pltpu.VMEM((TOK_PER_W,), jnp.int32),
            pltpu.VMEM((NBUF, CHUNK, D), jnp.float32),
            pltpu.VMEM((NBUF, CHUNK, D), jnp.float32),
            pltpu.VMEM((NBUF, CHUNK, D), jnp.float32),
        ] + [pltpu.SemaphoreType.DMA] * (3 * NBUF),
    )(_sc_body)


def kernel(source, word_table):
    idx = source[..., 0].astype(jnp.int32)
    idx3 = idx.reshape(B, R, CL)
    pe_idx = _pe_index_call(idx3)
    pe_pad = jnp.asarray(_PE_PAD)
    out = _sc_call()(word_table, pe_pad, idx.reshape(N), pe_idx.reshape(N))
    return out.reshape(B, SEG, SL, D)


# speedup vs baseline: 1.0058x; 1.0058x over previous
import functools
import math

import jax
import jax.numpy as jnp
import numpy as np
from jax import lax
from jax.experimental import pallas as pl
from jax.experimental.pallas import tpu as pltpu
from jax.experimental.pallas import tpu_sc as plsc

MAX_LEN = 5000
B, SEG, SL = 32, 16, 256
V, D = 100000, 128
PAD = 0

N = B * SEG * SL
S = SEG * SL
R, CL = 32, 128
SCALE = math.sqrt(D)

NC, NS, L = 2, 16, 16
NW = NC * NS
TOK_PER_W = N // NW
CHUNK = 64
NBUF = 4
NCHUNK = TOK_PER_W // CHUNK

PE_ROWS = 4224
PE_ZERO = S


def _build_pe() -> np.ndarray:
    pe = np.zeros((PE_ROWS, D), dtype=np.float32)
    position = np.arange(0, S, dtype=np.float32)[:, None]
    div_term = np.exp(
        np.arange(0, D, 2, dtype=np.float32) * -(math.log(10000.0) / D))
    pe[:S, 0::2] = np.sin(position * div_term)
    pe[:S, 1::2] = np.cos(position * div_term)
    return pe


_PE_PAD = _build_pe()


def _incl_cumsum(x, axis):
    n = x.shape[axis]
    k = 1
    while k < n:
        shape = list(x.shape)
        shape[axis] = k
        z = jnp.zeros(shape, x.dtype)
        shifted = jnp.concatenate(
            [z, lax.slice_in_dim(x, 0, n - k, axis=axis)], axis=axis)
        x = x + shifted
        k *= 2
    return x


def _pe_index_body(idx_ref, out_ref):
    idx = idx_ref[...]
    m = (idx != PAD).astype(jnp.int32)

    inc_b = _incl_cumsum(m, axis=0)
    p_excl = inc_b - m
    cc = lax.index_in_dim(inc_b, B - 1, axis=0, keepdims=False)

    inc_c = _incl_cumsum(cc, axis=1)
    row_tot = lax.slice_in_dim(inc_c, CL - 1, CL, axis=1)
    inc_r = _incl_cumsum(row_tot, axis=0)
    s_excl = (inc_r - row_tot) + (inc_c - cc)

    rank = p_excl + s_excl[None, :, :]

    seq = jnp.sum(jnp.sum(m, axis=2), axis=1, keepdims=True)
    c_incl = _incl_cumsum(seq, axis=0)

    c_at = jnp.zeros_like(rank)
    for j in range(B):
        cj = lax.slice(c_incl, (j, 0), (j + 1, 1)).reshape(1, 1, 1)
        dj = lax.slice(seq, (j, 0), (j + 1, 1)).reshape(1, 1, 1)
        c_at = c_at + jnp.where(rank >= cj, dj, 0)

    offset = rank - c_at
    out_ref[...] = jnp.where(idx != PAD, offset, PE_ZERO)


_pe_index_call = pl.pallas_call(
    _pe_index_body,
    out_shape=jax.ShapeDtypeStruct((B, R, CL), jnp.int32),
)


def _compute_chunk(wr, pr, ob):
    def row(r, c):
        for j in range(D // L):
            w = wr[r, pl.ds(j * L, L)]
            p = pr[r, pl.ds(j * L, L)]
            ob[r, pl.ds(j * L, L)] = w * SCALE + p
        return c

    lax.fori_loop(0, CHUNK, row, 0, unroll=4)


def _sc_body(word_hbm, pe_hbm, idx_hbm, pidx_hbm, out_hbm,
             idx_all, pidx_all, wrows, prows, obuf, *sems):
    sem_w = sems[0:NBUF]
    sem_p = sems[NBUF:2 * NBUF]
    sem_o = sems[2 * NBUF:3 * NBUF]

    wid = lax.axis_index("s") * NC + lax.axis_index("c")
    base = pl.multiple_of(wid * TOK_PER_W, TOK_PER_W)
    pltpu.sync_copy(idx_hbm.at[pl.ds(base, TOK_PER_W)], idx_all)
    pltpu.sync_copy(pidx_hbm.at[pl.ds(base, TOK_PER_W)], pidx_all)

    def start_gathers(g, b):
        off = pl.multiple_of(g * CHUNK, CHUNK)
        pltpu.async_copy(
            word_hbm.at[idx_all.at[pl.ds(off, CHUNK)]], wrows.at[b], sem_w[b])
        pltpu.async_copy(
            pe_hbm.at[pidx_all.at[pl.ds(off, CHUNK)]], prows.at[b], sem_p[b])

    def wait_gathers(b):
        pltpu.make_async_copy(
            word_hbm.at[idx_all.at[pl.ds(0, CHUNK)]], wrows.at[b],
            sem_w[b]).wait()
        pltpu.make_async_copy(
            pe_hbm.at[pidx_all.at[pl.ds(0, CHUNK)]], prows.at[b],
            sem_p[b]).wait()

    def start_out(g, b):
        off = pl.multiple_of(base + g * CHUNK, CHUNK)
        pltpu.async_copy(obuf.at[b], out_hbm.at[pl.ds(off, CHUNK)], sem_o[b])

    def wait_out(b):
        pltpu.make_async_copy(
            obuf.at[b], out_hbm.at[pl.ds(0, CHUNK)], sem_o[b]).wait()

    for b in range(NBUF):
        start_gathers(b, b)

    def group(k, c):
        g0 = k * NBUF
        for b in range(NBUF):
            g = g0 + b
            wait_gathers(b)

            @pl.when(g >= NBUF)
            def _():
                wait_out(b)

            _compute_chunk(wrows.at[b], prows.at[b], obuf.at[b])
            start_out(g, b)

            @pl.when(g + NBUF < NCHUNK)
            def _():
                start_gathers(g + NBUF, b)

        return c

    lax.fori_loop(0, NCHUNK // NBUF, group, 0)
    for b in range(NBUF):
        wait_out(b)


@functools.cache
def _sc_call():
    return functools.partial(
        pl.kernel,
        out_type=jax.ShapeDtypeStruct((N, D), jnp.float32),
        mesh=plsc.VectorSubcoreMesh(
            core_axis_name="c", subcore_axis_name="s",
            num_cores=NC, num_subcores=NS),
        scratch_types=[
            pltpu.VMEM((TOK_PER_W,), jnp.int32),
            pltpu.VMEM((TOK_PER_W,), jnp.int32),
            pltpu.VMEM((NBUF, CHUNK, D), jnp.float32),
            pltpu.VMEM((NBUF, CHUNK, D), jnp.float32),
            pltpu.VMEM((NBUF, CHUNK, D), jnp.float32),
        ] + [pltpu.SemaphoreType.DMA] * (3 * NBUF),
    )(_sc_body)


def kernel(source, word_table):
    idx = source[..., 0].astype(jnp.int32)
    idx3 = idx.reshape(B, R, CL)
    pe_idx = _pe_index_call(idx3)
    pe_pad = jnp.asarray(_PE_PAD)
    out = _sc_call()(word_table, pe_pad, idx.reshape(N), pe_idx.reshape(N))
    return out.reshape(B, SEG, SL, D)
